# baseline (speedup 1.0000x reference)
import functools

import jax
import jax.numpy as jnp
from jax import lax
from jax.experimental import pallas as pl
from jax.experimental.pallas import tpu as pltpu
from jax.experimental.pallas import tpu_sc as plsc

N_SRC = 10000
N_DST = 10000
CH = 128
E = 320000
B = 64
NCHUNK = E // B
NC = 2
NS = 16
NW = NC * NS
ROWS_PAD = 10240
ZCH = B
NZ = ROWS_PAD // NS // ZCH
KMAX = (NCHUNK + NW - 1) // NW
NSEC = ((KMAX + 2 + 3) // 4) * 4

_mesh = plsc.VectorSubcoreMesh(
    core_axis_name="c", subcore_axis_name="s", num_cores=NC, num_subcores=NS)


@functools.partial(
    pl.kernel,
    out_type=(
        jax.ShapeDtypeStruct((NC, ROWS_PAD, CH), jnp.float32),
        jax.ShapeDtypeStruct((NW, ROWS_PAD), jnp.float32),
    ),
    mesh=_mesh,
    scratch_types=[
        pltpu.VMEM_SHARED((ROWS_PAD, CH), jnp.float32),
        pltpu.VMEM((4, B), jnp.int32),
        pltpu.VMEM((4, B), jnp.int32),
        pltpu.VMEM((4, B, CH), jnp.float32),
        pltpu.VMEM((ROWS_PAD,), jnp.float32),
        [pltpu.SemaphoreType.DMA] * 4,
        [pltpu.SemaphoreType.DMA] * 4,
        [pltpu.SemaphoreType.DMA] * 4,
    ],
    compiler_params=pltpu.CompilerParams(needs_layout_passes=False),
)
def _sc_aggregate(x_hbm, src_hbm, dst_hbm, out_hbm, cnt_hbm,
                  acc, idx_s, idx_d, rows, cnt, semi, semg, sems):
    c = lax.axis_index("c")
    s = lax.axis_index("s")
    wid = s * NC + c

    zeros16 = jnp.zeros((16,), jnp.float32)

    def zrow(r, carry):
        for j in range(CH // 16):
            rows[0, r, pl.ds(j * 16, 16)] = zeros16
        return carry
    lax.fori_loop(0, B, zrow, 0)

    def zcnt(r, carry):
        cnt[pl.ds(r * 16, 16)] = zeros16
        return carry
    lax.fori_loop(0, ROWS_PAD // 16, zcnt, 0)

    for j in range(NZ):
        r0 = s * (ROWS_PAD // NS) + j * ZCH
        pltpu.sync_copy(rows.at[0], acc.at[pl.ds(r0, ZCH), :])
    plsc.subcore_barrier()

    ones16 = jnp.ones((16,), jnp.float32)

    def issue_idx(b, kk):
        base = (wid + kk * NW) * B
        pltpu.async_copy(src_hbm.at[pl.ds(base, B)], idx_s.at[b], semi[b])
        pltpu.async_copy(dst_hbm.at[pl.ds(base, B)], idx_d.at[b], semi[b])

    def wait_idx(b):
        pltpu.make_async_copy(src_hbm.at[pl.ds(0, B)], idx_s.at[b],
                              semi[b]).wait()
        pltpu.make_async_copy(dst_hbm.at[pl.ds(0, B)], idx_d.at[b],
                              semi[b]).wait()

    def issue_gather(b):
        pltpu.async_copy(x_hbm.at[idx_s.at[b]], rows.at[b], semg[b])

    def wait_gather(b):
        pltpu.make_async_copy(x_hbm.at[idx_s.at[b]], rows.at[b],
                              semg[b]).wait()

    def issue_scat(b):
        pltpu.async_copy(rows.at[b], acc.at[idx_d.at[b]], sems[b], add=True)

    def wait_scat(b):
        pltpu.make_async_copy(rows.at[b], acc.at[idx_d.at[b]],
                              sems[b]).wait()

    def section(b, kk):
        chunk = wid + kk * NW
        nb = (b + 1) % 4
        n2 = (b + 2) % 4

        @pl.when(chunk < NCHUNK)
        def _():
            wait_gather(b)
            issue_scat(b)
            for j in range(B // 16):
                d16 = idx_d[b, pl.ds(j * 16, 16)]
                plsc.addupdate_scatter(cnt, [d16], ones16)

        @pl.when(chunk + NW < NCHUNK)
        def _():
            wait_idx(nb)
            issue_gather(nb)

        if_kk_ge_2 = jnp.logical_and(kk >= 2, wid + (kk - 2) * NW < NCHUNK)

        @pl.when(if_kk_ge_2)
        def _():
            wait_scat(n2)

        @pl.when(chunk + 2 * NW < NCHUNK)
        def _():
            issue_idx(n2, kk + 2)

    issue_idx(0, 0)
    wait_idx(0)
    issue_gather(0)

    @pl.when(wid + NW < NCHUNK)
    def _():
        issue_idx(1, 1)

    def outer(t, carry):
        for u in range(4):
            section(u, 4 * t + u)
        return carry
    lax.fori_loop(0, NSEC // 4, outer, 0)

    pltpu.sync_copy(cnt, cnt_hbm.at[wid])

    plsc.subcore_barrier()

    for j in range(NZ):
        r0 = s * (ROWS_PAD // NS) + j * ZCH
        pltpu.sync_copy(acc.at[pl.ds(r0, ZCH), :], rows.at[0])
        pltpu.sync_copy(rows.at[0], out_hbm.at[c, pl.ds(r0, ZCH), :])


def _finish_body(a_ref, c_ref, w_ref, b_ref, o_ref):
    a = a_ref[0] + a_ref[1]
    cnt = jnp.sum(c_ref[...], axis=0)[:, None]
    y = jnp.dot(a, w_ref[...], preferred_element_type=jnp.float32)
    y = y * lax.rsqrt(1.0 + cnt) + b_ref[...]
    o_ref[...] = jnp.maximum(y, 0.0)


RBLK = 512

_finish = pl.pallas_call(
    _finish_body,
    grid=(ROWS_PAD // RBLK,),
    in_specs=[
        pl.BlockSpec((NC, RBLK, CH), lambda i: (0, i, 0)),
        pl.BlockSpec((NW, RBLK), lambda i: (0, i)),
        pl.BlockSpec((CH, CH), lambda i: (0, 0)),
        pl.BlockSpec((1, CH), lambda i: (0, 0)),
    ],
    out_specs=pl.BlockSpec((RBLK, CH), lambda i: (i, 0)),
    out_shape=jax.ShapeDtypeStruct((ROWS_PAD, CH), jnp.float32),
)


def kernel(x, edge_index, W, b):
    x = x.astype(jnp.float32)
    src = edge_index[0].astype(jnp.int32)
    dst = edge_index[1].astype(jnp.int32)
    partials, counts = _sc_aggregate(x, src, dst)
    out = _finish(partials, counts, W.astype(jnp.float32),
                  b.astype(jnp.float32).reshape(1, CH))
    return out[:N_DST]


# speedup vs baseline: 1.2781x; 1.2781x over previous
import functools

import jax
import jax.numpy as jnp
from jax import lax
from jax.experimental import pallas as pl
from jax.experimental.pallas import tpu as pltpu
from jax.experimental.pallas import tpu_sc as plsc

N_SRC = 10000
N_DST = 10000
CH = 128
E = 320000
B = 128
NCHUNK = E // B
NC = 2
NS = 16
NW = NC * NS
ROWS_PAD = 10240
ZCH = B
NZ = ROWS_PAD // NS // ZCH
KMAX = (NCHUNK + NW - 1) // NW
NSEC = ((KMAX + 1 + 3) // 4) * 4

_mesh = plsc.VectorSubcoreMesh(
    core_axis_name="c", subcore_axis_name="s", num_cores=NC, num_subcores=NS)


@functools.partial(
    pl.kernel,
    out_type=(
        jax.ShapeDtypeStruct((NC, ROWS_PAD, CH), jnp.float32),
        jax.ShapeDtypeStruct((NW, ROWS_PAD), jnp.float32),
    ),
    mesh=_mesh,
    scratch_types=[
        pltpu.VMEM_SHARED((ROWS_PAD, CH), jnp.float32),
        pltpu.VMEM((4, B), jnp.int32),
        pltpu.VMEM((4, B), jnp.int32),
        pltpu.VMEM((2, B, CH), jnp.float32),
        pltpu.VMEM((ROWS_PAD,), jnp.float32),
        [pltpu.SemaphoreType.DMA] * 4,
        [pltpu.SemaphoreType.DMA] * 2,
        [pltpu.SemaphoreType.DMA] * 2,
    ],
    compiler_params=pltpu.CompilerParams(needs_layout_passes=False),
)
def _sc_aggregate(x_hbm, src_hbm, dst_hbm, out_hbm, cnt_hbm,
                  acc, idx_s, idx_d, rows, cnt, semi, semg, sems):
    c = lax.axis_index("c")
    s = lax.axis_index("s")
    wid = s * NC + c

    zeros16 = jnp.zeros((16,), jnp.float32)

    def zrow(r, carry):
        for j in range(CH // 16):
            rows[0, r, pl.ds(j * 16, 16)] = zeros16
        return carry
    lax.fori_loop(0, B, zrow, 0)

    def zcnt(r, carry):
        cnt[pl.ds(r * 16, 16)] = zeros16
        return carry
    lax.fori_loop(0, ROWS_PAD // 16, zcnt, 0)

    for j in range(NZ):
        r0 = s * (ROWS_PAD // NS) + j * ZCH
        pltpu.sync_copy(rows.at[0], acc.at[pl.ds(r0, ZCH), :])
    plsc.subcore_barrier()

    ones16 = jnp.ones((16,), jnp.float32)

    def issue_idx(b, kk):
        base = (wid + kk * NW) * B
        pltpu.async_copy(src_hbm.at[pl.ds(base, B)], idx_s.at[b], semi[b])
        pltpu.async_copy(dst_hbm.at[pl.ds(base, B)], idx_d.at[b], semi[b])

    def wait_idx(b):
        pltpu.make_async_copy(src_hbm.at[pl.ds(0, B)], idx_s.at[b],
                              semi[b]).wait()
        pltpu.make_async_copy(dst_hbm.at[pl.ds(0, B)], idx_d.at[b],
                              semi[b]).wait()

    def issue_gather(b2, b4):
        pltpu.async_copy(x_hbm.at[idx_s.at[b4]], rows.at[b2], semg[b2])

    def wait_gather(b2, b4):
        pltpu.make_async_copy(x_hbm.at[idx_s.at[b4]], rows.at[b2],
                              semg[b2]).wait()

    def issue_scat(b2, b4):
        pltpu.async_copy(rows.at[b2], acc.at[idx_d.at[b4]], sems[b2],
                         add=True)

    def wait_scat(b2, b4):
        pltpu.make_async_copy(rows.at[b2], acc.at[idx_d.at[b4]],
                              sems[b2]).wait()

    def section(u, kk):
        chunk = wid + kk * NW
        b2, p2 = u % 2, (u + 1) % 2
        b4, p4, n4, f4 = u, (u + 3) % 4, (u + 1) % 4, (u + 2) % 4

        @pl.when(chunk < NCHUNK)
        def _():
            wait_gather(b2, b4)
            issue_scat(b2, b4)

        @pl.when(jnp.logical_and(kk >= 1, wid + (kk - 1) * NW < NCHUNK))
        def _():
            wait_scat(p2, p4)

        @pl.when(chunk + NW < NCHUNK)
        def _():
            wait_idx(n4)
            issue_gather(p2, n4)

        @pl.when(chunk + 2 * NW < NCHUNK)
        def _():
            issue_idx(f4, kk + 2)

        @pl.when(chunk < NCHUNK)
        def _():
            for j in range(B // 16):
                d16 = idx_d[b4, pl.ds(j * 16, 16)]
                plsc.addupdate_scatter(cnt, [d16], ones16)

    issue_idx(0, 0)
    wait_idx(0)
    issue_gather(0, 0)

    @pl.when(wid + NW < NCHUNK)
    def _():
        issue_idx(1, 1)

    def outer(t, carry):
        for u in range(4):
            section(u, 4 * t + u)
        return carry
    lax.fori_loop(0, NSEC // 4, outer, 0)

    pltpu.sync_copy(cnt, cnt_hbm.at[wid])

    plsc.subcore_barrier()

    for j in range(NZ):
        r0 = s * (ROWS_PAD // NS) + j * ZCH
        pltpu.sync_copy(acc.at[pl.ds(r0, ZCH), :], rows.at[0])
        pltpu.sync_copy(rows.at[0], out_hbm.at[c, pl.ds(r0, ZCH), :])


def _finish_body(a_ref, c_ref, w_ref, b_ref, o_ref):
    a = a_ref[0] + a_ref[1]
    cnt = jnp.sum(c_ref[...], axis=0)[:, None]
    y = jnp.dot(a, w_ref[...], preferred_element_type=jnp.float32)
    y = y * lax.rsqrt(1.0 + cnt) + b_ref[...]
    o_ref[...] = jnp.maximum(y, 0.0)


RBLK = 512

_finish = pl.pallas_call(
    _finish_body,
    grid=(ROWS_PAD // RBLK,),
    in_specs=[
        pl.BlockSpec((NC, RBLK, CH), lambda i: (0, i, 0)),
        pl.BlockSpec((NW, RBLK), lambda i: (0, i)),
        pl.BlockSpec((CH, CH), lambda i: (0, 0)),
        pl.BlockSpec((1, CH), lambda i: (0, 0)),
    ],
    out_specs=pl.BlockSpec((RBLK, CH), lambda i: (i, 0)),
    out_shape=jax.ShapeDtypeStruct((ROWS_PAD, CH), jnp.float32),
)


def kernel(x, edge_index, W, b):
    x = x.astype(jnp.float32)
    src = edge_index[0].astype(jnp.int32)
    dst = edge_index[1].astype(jnp.int32)
    partials, counts = _sc_aggregate(x, src, dst)
    out = _finish(partials, counts, W.astype(jnp.float32),
                  b.astype(jnp.float32).reshape(1, CH))
    return out[:N_DST]


# speedup vs baseline: 1.3281x; 1.0391x over previous
import functools

import jax
import jax.numpy as jnp
from jax import lax
from jax.experimental import pallas as pl
from jax.experimental.pallas import tpu as pltpu
from jax.experimental.pallas import tpu_sc as plsc

N_SRC = 10000
N_DST = 10000
CH = 128
E = 320000
B = 128
NCHUNK = E // B
NC = 2
NS = 16
NW = NC * NS
ROWS_PAD = 10240
ZCH = ROWS_PAD // NS // 5
KMAX = (NCHUNK + NW - 1) // NW

_mesh = plsc.VectorSubcoreMesh(
    core_axis_name="c", subcore_axis_name="s", num_cores=NC, num_subcores=NS)


@functools.partial(
    pl.kernel,
    out_type=(
        jax.ShapeDtypeStruct((NC, ROWS_PAD, CH), jnp.float32),
        jax.ShapeDtypeStruct((NW, ROWS_PAD), jnp.float32),
    ),
    mesh=_mesh,
    scratch_types=[
        pltpu.VMEM_SHARED((ROWS_PAD, CH), jnp.float32),
        pltpu.VMEM((2, B), jnp.int32),
        pltpu.VMEM((2, B), jnp.int32),
        pltpu.VMEM((2, B, CH), jnp.float32),
        pltpu.VMEM((ROWS_PAD,), jnp.float32),
        pltpu.SemaphoreType.DMA,
        pltpu.SemaphoreType.DMA,
        pltpu.SemaphoreType.DMA,
        pltpu.SemaphoreType.DMA,
    ],
    compiler_params=pltpu.CompilerParams(needs_layout_passes=False),
)
def _sc_aggregate(x_hbm, src_hbm, dst_hbm, out_hbm, cnt_hbm,
                  acc, idx_s, idx_d, rows, cnt,
                  semi0, semi1, semg0, semg1):
    c = lax.axis_index("c")
    s = lax.axis_index("s")
    wid = s * NC + c
    semi = (semi0, semi1)
    semg = (semg0, semg1)

    zeros16 = jnp.zeros((16,), jnp.float32)

    def zrow(r, carry):
        for j in range(CH // 16):
            rows[0, r, pl.ds(j * 16, 16)] = zeros16
        return carry
    lax.fori_loop(0, B, zrow, 0)

    def zcnt(r, carry):
        cnt[pl.ds(r * 16, 16)] = zeros16
        return carry
    lax.fori_loop(0, ROWS_PAD // 16, zcnt, 0)

    for j in range(ROWS_PAD // NS // ZCH):
        r0 = s * (ROWS_PAD // NS) + j * ZCH
        pltpu.sync_copy(rows.at[0], acc.at[pl.ds(r0, ZCH), :])
    plsc.subcore_barrier()

    ones16 = jnp.ones((16,), jnp.float32)

    def issue_idx(b, kk):
        base = (wid + kk * NW) * B
        pltpu.async_copy(src_hbm.at[pl.ds(base, B)], idx_s.at[b], semi[b])
        pltpu.async_copy(dst_hbm.at[pl.ds(base, B)], idx_d.at[b], semi[b])

    def wait_idx(b):
        pltpu.make_async_copy(src_hbm.at[pl.ds(0, B)], idx_s.at[b],
                              semi[b]).wait()
        pltpu.make_async_copy(dst_hbm.at[pl.ds(0, B)], idx_d.at[b],
                              semi[b]).wait()

    def issue_gather(b):
        pltpu.async_copy(x_hbm.at[idx_s.at[b]], rows.at[b], semg[b])

    def wait_gather(b):
        pltpu.make_async_copy(x_hbm.at[idx_s.at[b]], rows.at[b],
                              semg[b]).wait()

    def section(b, kk, chunk):
        @pl.when(chunk < NCHUNK)
        def _():
            nb = 1 - b

            @pl.when(chunk + NW < NCHUNK)
            def _():
                wait_idx(nb)
                issue_gather(nb)
            for j in range(B // 16):
                d16 = idx_d[b, pl.ds(j * 16, 16)]
                plsc.addupdate_scatter(cnt, [d16], ones16)
            wait_gather(b)
            pltpu.sync_copy(rows.at[b], acc.at[idx_d.at[b]], add=True)

            @pl.when(chunk + 2 * NW < NCHUNK)
            def _():
                issue_idx(b, kk + 2)

    issue_idx(0, 0)
    wait_idx(0)
    issue_gather(0)

    @pl.when(wid + NW < NCHUNK)
    def _():
        issue_idx(1, 1)

    def outer(t, carry):
        kk0 = 2 * t
        section(0, kk0, wid + kk0 * NW)
        section(1, kk0 + 1, wid + (kk0 + 1) * NW)
        return carry
    lax.fori_loop(0, (KMAX + 1) // 2, outer, 0)

    pltpu.sync_copy(cnt, cnt_hbm.at[wid])

    plsc.subcore_barrier()

    for j in range(ROWS_PAD // NS // ZCH):
        r0 = s * (ROWS_PAD // NS) + j * ZCH
        pltpu.sync_copy(acc.at[pl.ds(r0, ZCH), :], rows.at[0])
        pltpu.sync_copy(rows.at[0], out_hbm.at[c, pl.ds(r0, ZCH), :])


def _finish_body(a_ref, c_ref, w_ref, b_ref, o_ref):
    a = a_ref[0] + a_ref[1]
    cnt = jnp.sum(c_ref[...], axis=0)[:, None]
    y = jnp.dot(a, w_ref[...], preferred_element_type=jnp.float32)
    y = y * lax.rsqrt(1.0 + cnt) + b_ref[...]
    o_ref[...] = jnp.maximum(y, 0.0)


RBLK = 512

_finish = pl.pallas_call(
    _finish_body,
    grid=(ROWS_PAD // RBLK,),
    in_specs=[
        pl.BlockSpec((NC, RBLK, CH), lambda i: (0, i, 0)),
        pl.BlockSpec((NW, RBLK), lambda i: (0, i)),
        pl.BlockSpec((CH, CH), lambda i: (0, 0)),
        pl.BlockSpec((1, CH), lambda i: (0, 0)),
    ],
    out_specs=pl.BlockSpec((RBLK, CH), lambda i: (i, 0)),
    out_shape=jax.ShapeDtypeStruct((ROWS_PAD, CH), jnp.float32),
)


def kernel(x, edge_index, W, b):
    x = x.astype(jnp.float32)
    src = edge_index[0].astype(jnp.int32)
    dst = edge_index[1].astype(jnp.int32)
    partials, counts = _sc_aggregate(x, src, dst)
    out = _finish(partials, counts, W.astype(jnp.float32),
                  b.astype(jnp.float32).reshape(1, CH))
    return out[:N_DST]
